# baseline (speedup 1.0000x reference)
import functools
import math

import numpy as np
import jax
import jax.numpy as jnp
from jax import lax
from jax.experimental import pallas as pl
from jax.experimental.pallas import tpu as pltpu
from jax.experimental.pallas import tpu_sc as plsc

_HEADS = 8
_KD = 512
_HALF = 256
_NK = 512
_KNN = 32
_IN = 1024
_VD = 256
_BS = 2048
_TB = 256
_NB = _BS // _TB
_NCAND = 128
_NW = 32
_TPW = _BS // _NW


def _build_cand_tables():
    ci, cj = [], []
    for i in range(_KNN):
        for j in range(min(_KNN, _KNN // (i + 1))):
            ci.append(i)
            cj.append(j)
    n = len(ci)
    assert n <= _NCAND
    p1 = np.zeros((_KNN, _NCAND), np.float32)
    p2 = np.zeros((_KNN, _NCAND), np.float32)
    for c in range(n):
        p1[ci[c], c] = 1.0
        p2[cj[c], c] = 1.0
    bias = np.zeros((1, _NCAND), np.float32)
    bias[0, n:] = -1e30
    return p1, p2, bias


_P1, _P2, _BIAS = _build_cand_tables()


def _topk32(s, src=None):
    r, c = s.shape
    io = lax.broadcasted_iota(jnp.int32, (r, c), 1)
    lane = lax.broadcasted_iota(jnp.int32, (r, _KNN), 1)
    outv = jnp.zeros((r, _KNN), jnp.float32)
    outi = jnp.zeros((r, _KNN), jnp.float32)
    neg = jnp.float32(float("-inf"))
    for k in range(_KNN):
        m = jnp.max(s, axis=1, keepdims=True)
        am = jnp.min(jnp.where(s == m, io, c), axis=1, keepdims=True)
        hit = io == am
        if src is None:
            pick = am.astype(jnp.float32)
        else:
            pick = jnp.sum(jnp.where(hit, src, 0.0), axis=1, keepdims=True)
        outv = jnp.where(lane == k, m, outv)
        outi = jnp.where(lane == k, pick, outi)
        s = jnp.where(hit, neg, s)
    return outv, outi


def _tc_body(x_ref, wq_ref, bq_ref, keys_ref, p1_ref, p2_ref, bias_ref,
             sc_out_ref, idx_out_ref):
    hi = lax.Precision.HIGHEST
    q = jnp.dot(x_ref[...], wq_ref[...],
                preferred_element_type=jnp.float32)
    q = q + bq_ref[...]
    dn = (((1,), (1,)), ((), ()))
    p1 = p1_ref[...]
    p2 = p2_ref[...]
    for h in range(_HEADS):
        qh = q[:, h * _KD:(h + 1) * _KD]
        s1 = lax.dot_general(qh[:, :_HALF], keys_ref[h, 0], dn,
                             preferred_element_type=jnp.float32)
        s2 = lax.dot_general(qh[:, _HALF:], keys_ref[h, 1], dn,
                             preferred_element_type=jnp.float32)
        v1, i1 = _topk32(s1)
        v2, i2 = _topk32(s2)
        cs = jnp.dot(v1, p1, precision=hi, preferred_element_type=jnp.float32) \
            + jnp.dot(v2, p2, precision=hi, preferred_element_type=jnp.float32) \
            + bias_ref[...]
        cidx = jnp.dot(i1, p1, precision=hi,
                       preferred_element_type=jnp.float32) * jnp.float32(_NK) \
            + jnp.dot(i2, p2, precision=hi, preferred_element_type=jnp.float32)
        sv, si = _topk32(cs, src=cidx)
        e = jnp.exp(sv - sv[:, :1])
        sm = e / jnp.sum(e, axis=1, keepdims=True)
        sc_out_ref[:, h * _KNN:(h + 1) * _KNN] = sm
        idx_out_ref[:, h * _KNN:(h + 1) * _KNN] = \
            (si + jnp.float32(0.5)).astype(jnp.int32)


def _tc_topk(x, w_q, bq2, keys, p1, p2, bias):
    f32 = jnp.float32
    return pl.pallas_call(
        _tc_body,
        grid=(_NB,),
        in_specs=[
            pl.BlockSpec((_TB, _IN), lambda b: (b, 0)),
            pl.BlockSpec((_IN, _HEADS * _KD), lambda b: (0, 0)),
            pl.BlockSpec((1, _HEADS * _KD), lambda b: (0, 0)),
            pl.BlockSpec((_HEADS, 2, _NK, _HALF), lambda b: (0, 0, 0, 0)),
            pl.BlockSpec((_KNN, _NCAND), lambda b: (0, 0)),
            pl.BlockSpec((_KNN, _NCAND), lambda b: (0, 0)),
            pl.BlockSpec((1, _NCAND), lambda b: (0, 0)),
        ],
        out_specs=[
            pl.BlockSpec((_TB, _HEADS * _KNN), lambda b: (b, 0)),
            pl.BlockSpec((_TB, _HEADS * _KNN), lambda b: (b, 0)),
        ],
        out_shape=[
            jax.ShapeDtypeStruct((_BS, _HEADS * _KNN), f32),
            jax.ShapeDtypeStruct((_BS, _HEADS * _KNN), jnp.int32),
        ],
    )(x, w_q, bq2, keys, p1, p2, bias)


def _acc_chunk(buf, acc):
    def body(r, a):
        r4 = r * 4
        for rr in range(4):
            a = tuple(
                a[c] + buf[r4 + rr, pl.ds(c * 16, 16)] for c in range(16)
            )
        return a
    return lax.fori_loop(0, 128 // 4, body, acc)


def _sc_body(values_hbm, idx_hbm, sc_hbm, out_hbm, idx_v, sc_v, buf0, buf1,
             res_v, sem0, sem1):
    nc = 2
    wid = lax.axis_index("s") * nc + lax.axis_index("c")
    base = wid * _TPW
    pltpu.sync_copy(idx_hbm.at[pl.ds(base * 2, _TPW * 2)], idx_v)
    pltpu.sync_copy(sc_hbm.at[pl.ds(base, _TPW)], sc_v)
    pltpu.async_copy(values_hbm.at[idx_v.at[0]], buf0, sem0)

    @pl.loop(0, _TPW)
    def _token(t):
        g = t * 2
        pltpu.async_copy(values_hbm.at[idx_v.at[g + 1]], buf1, sem1)
        pltpu.make_async_copy(values_hbm.at[idx_v.at[g]], buf0, sem0).wait()
        acc0 = tuple(jnp.zeros((16,), jnp.float32) for _ in range(16))
        acc = _acc_chunk(buf0, acc0)

        @pl.when(t < _TPW - 1)
        def _():
            pltpu.async_copy(values_hbm.at[idx_v.at[g + 2]], buf0, sem0)

        pltpu.make_async_copy(values_hbm.at[idx_v.at[g + 1]], buf1, sem1).wait()
        acc = _acc_chunk(buf1, acc)
        for c in range(16):
            res_v[t, pl.ds(c * 16, 16)] = acc[c] * sc_v[t, pl.ds(c * 16, 16)]

    pltpu.sync_copy(res_v, out_hbm.at[pl.ds(base, _TPW)])


def _sc_gather(values, idx, scores):
    mesh = plsc.VectorSubcoreMesh(core_axis_name="c", subcore_axis_name="s")
    f32 = jnp.float32
    call = pl.kernel(
        _sc_body,
        out_type=jax.ShapeDtypeStruct((_BS, _VD), f32),
        mesh=mesh,
        scratch_types=[
            pltpu.VMEM((_TPW * 2, 128), jnp.int32),
            pltpu.VMEM((_TPW, _VD), f32),
            pltpu.VMEM((128, _VD), f32),
            pltpu.VMEM((128, _VD), f32),
            pltpu.VMEM((_TPW, _VD), f32),
            pltpu.SemaphoreType.DMA,
            pltpu.SemaphoreType.DMA,
        ],
        compiler_params=pltpu.CompilerParams(use_tc_tiling_on_sc=False),
    )
    return call(values, idx, scores)


def kernel(x, W_q, b_q, keys, values):
    bq2 = b_q.reshape(1, _HEADS * _KD)
    p1 = jnp.asarray(_P1)
    p2 = jnp.asarray(_P2)
    bias = jnp.asarray(_BIAS)
    sc, idx = _tc_topk(x, W_q, bq2, keys, p1, p2, bias)
    return _sc_gather(values, idx.reshape(_BS * 2, 128), sc)


# speedup vs baseline: 1.2611x; 1.2611x over previous
import functools
import math

import numpy as np
import jax
import jax.numpy as jnp
from jax import lax
from jax.experimental import pallas as pl
from jax.experimental.pallas import tpu as pltpu
from jax.experimental.pallas import tpu_sc as plsc

_HEADS = 8
_KD = 512
_HALF = 256
_NK = 512
_KNN = 32
_IN = 1024
_VD = 256
_BS = 2048
_TB = 256
_NB = _BS // _TB
_NCAND = 128
_NW = 32
_TPW = _BS // _NW


def _build_cand_tables():
    ci, cj = [], []
    for i in range(_KNN):
        for j in range(min(_KNN, _KNN // (i + 1))):
            ci.append(i)
            cj.append(j)
    n = len(ci)
    assert n <= _NCAND
    p1 = np.zeros((_KNN, _NCAND), np.float32)
    p2 = np.zeros((_KNN, _NCAND), np.float32)
    for c in range(n):
        p1[ci[c], c] = 1.0
        p2[cj[c], c] = 1.0
    bias = np.zeros((1, _NCAND), np.float32)
    bias[0, n:] = -1e30
    return p1, p2, bias


_P1, _P2, _BIAS = _build_cand_tables()


def _topk32(s, src=None):
    r, c = s.shape
    io = lax.broadcasted_iota(jnp.int32, (r, c), 1)
    lane = lax.broadcasted_iota(jnp.int32, (r, _KNN), 1)
    outv = jnp.zeros((r, _KNN), jnp.float32)
    outi = jnp.zeros((r, _KNN), jnp.float32)
    neg = jnp.float32(float("-inf"))
    for k in range(_KNN):
        m = jnp.max(s, axis=1, keepdims=True)
        am = jnp.min(jnp.where(s == m, io, c), axis=1, keepdims=True)
        hit = io == am
        if src is None:
            pick = am.astype(jnp.float32)
        else:
            pick = jnp.sum(jnp.where(hit, src, 0.0), axis=1, keepdims=True)
        outv = jnp.where(lane == k, m, outv)
        outi = jnp.where(lane == k, pick, outi)
        s = jnp.where(hit, neg, s)
    return outv, outi


def _tc_body(x_ref, wq_ref, bq_ref, keys_ref, p1_ref, p2_ref, bias_ref,
             sc_out_ref, idx_out_ref, sc_scr, idx_scr):
    h = pl.program_id(1)
    hi = lax.Precision.HIGHEST
    q = jnp.dot(x_ref[...], wq_ref[...],
                preferred_element_type=jnp.float32)
    q = q + bq_ref[0]
    k1 = keys_ref[0, 0]
    k2 = keys_ref[0, 1]
    dn = (((1,), (1,)), ((), ()))
    s1 = lax.dot_general(q[:, :_HALF], k1, dn,
                         preferred_element_type=jnp.float32)
    s2 = lax.dot_general(q[:, _HALF:], k2, dn,
                         preferred_element_type=jnp.float32)
    v1, i1 = _topk32(s1)
    v2, i2 = _topk32(s2)
    p1 = p1_ref[...]
    p2 = p2_ref[...]
    cs = jnp.dot(v1, p1, precision=hi, preferred_element_type=jnp.float32) \
        + jnp.dot(v2, p2, precision=hi, preferred_element_type=jnp.float32) \
        + bias_ref[...]
    cidx = jnp.dot(i1, p1, precision=hi,
                   preferred_element_type=jnp.float32) * jnp.float32(_NK) \
        + jnp.dot(i2, p2, precision=hi, preferred_element_type=jnp.float32)
    sv, si = _topk32(cs, src=cidx)
    e = jnp.exp(sv - sv[:, :1])
    sm = e / jnp.sum(e, axis=1, keepdims=True)
    sc_scr[h] = sm
    idx_scr[h] = (si + jnp.float32(0.5)).astype(jnp.int32)

    @pl.when(h == _HEADS - 1)
    def _flush():
        sc_all = jnp.concatenate([sc_scr[i] for i in range(_HEADS)], axis=1)
        idx_all = jnp.concatenate([idx_scr[i] for i in range(_HEADS)], axis=1)
        sc_out_ref[...] = sc_all.reshape(2 * _TB, 128)
        idx_out_ref[...] = idx_all.reshape(2 * _TB, 128)


def _tc_topk(x, w_q, bq2, keys, p1, p2, bias):
    f32 = jnp.float32
    return pl.pallas_call(
        _tc_body,
        grid=(_NB, _HEADS),
        in_specs=[
            pl.BlockSpec((_TB, _IN), lambda b, h: (b, 0)),
            pl.BlockSpec((_IN, _KD), lambda b, h: (0, h)),
            pl.BlockSpec((1, 1, _KD), lambda b, h: (h, 0, 0)),
            pl.BlockSpec((1, 2, _NK, _HALF), lambda b, h: (h, 0, 0, 0)),
            pl.BlockSpec((_KNN, _NCAND), lambda b, h: (0, 0)),
            pl.BlockSpec((_KNN, _NCAND), lambda b, h: (0, 0)),
            pl.BlockSpec((1, _NCAND), lambda b, h: (0, 0)),
        ],
        out_specs=[
            pl.BlockSpec((2 * _TB, 128), lambda b, h: (b, 0)),
            pl.BlockSpec((2 * _TB, 128), lambda b, h: (b, 0)),
        ],
        out_shape=[
            jax.ShapeDtypeStruct((_BS * 2, 128), f32),
            jax.ShapeDtypeStruct((_BS * 2, 128), jnp.int32),
        ],
        scratch_shapes=[
            pltpu.VMEM((_HEADS, _TB, _KNN), f32),
            pltpu.VMEM((_HEADS, _TB, _KNN), jnp.int32),
        ],
    )(x, w_q, bq2, keys, p1, p2, bias)


def _acc_chunk(buf, acc):
    def body(r, a):
        r4 = r * 4
        for rr in range(4):
            a = tuple(
                a[c] + buf[r4 + rr, pl.ds(c * 16, 16)] for c in range(16)
            )
        return a
    return lax.fori_loop(0, 128 // 4, body, acc)


def _sc_body(values_hbm, idx_hbm, sc_hbm, out_hbm, idx_v, sc_v, buf0, buf1,
             res_v, sem0, sem1):
    nc = 2
    wid = lax.axis_index("s") * nc + lax.axis_index("c")
    base = wid * _TPW
    pltpu.sync_copy(idx_hbm.at[pl.ds(base * 2, _TPW * 2)], idx_v)
    pltpu.sync_copy(sc_hbm.at[pl.ds(base * 2, _TPW * 2)], sc_v)
    pltpu.async_copy(values_hbm.at[idx_v.at[0]], buf0, sem0)

    @pl.loop(0, _TPW)
    def _token(t):
        g = t * 2
        pltpu.async_copy(values_hbm.at[idx_v.at[g + 1]], buf1, sem1)
        pltpu.make_async_copy(values_hbm.at[idx_v.at[g]], buf0, sem0).wait()
        acc0 = tuple(jnp.zeros((16,), jnp.float32) for _ in range(16))
        acc = _acc_chunk(buf0, acc0)

        @pl.when(t < _TPW - 1)
        def _():
            pltpu.async_copy(values_hbm.at[idx_v.at[g + 2]], buf0, sem0)

        pltpu.make_async_copy(values_hbm.at[idx_v.at[g + 1]], buf1, sem1).wait()
        acc = _acc_chunk(buf1, acc)
        for c in range(16):
            res_v[t, pl.ds(c * 16, 16)] = \
                acc[c] * sc_v[g + c // 8, pl.ds((c % 8) * 16, 16)]

    pltpu.sync_copy(res_v, out_hbm.at[pl.ds(base, _TPW)])


def _sc_gather(values, idx, scores):
    mesh = plsc.VectorSubcoreMesh(core_axis_name="c", subcore_axis_name="s")
    f32 = jnp.float32
    call = pl.kernel(
        _sc_body,
        out_type=jax.ShapeDtypeStruct((_BS, _VD), f32),
        mesh=mesh,
        scratch_types=[
            pltpu.VMEM((_TPW * 2, 128), jnp.int32),
            pltpu.VMEM((_TPW * 2, 128), f32),
            pltpu.VMEM((128, _VD), f32),
            pltpu.VMEM((128, _VD), f32),
            pltpu.VMEM((_TPW, _VD), f32),
            pltpu.SemaphoreType.DMA,
            pltpu.SemaphoreType.DMA,
        ],
        compiler_params=pltpu.CompilerParams(use_tc_tiling_on_sc=False),
    )
    return call(values, idx, scores)


def kernel(x, W_q, b_q, keys, values):
    bq2 = b_q.reshape(_HEADS, 1, _KD)
    p1 = jnp.asarray(_P1)
    p2 = jnp.asarray(_P2)
    bias = jnp.asarray(_BIAS)
    sc, idx = _tc_topk(x, W_q, bq2, keys, p1, p2, bias)
    return _sc_gather(values, idx, sc)


# speedup vs baseline: 1.9049x; 1.5105x over previous
import functools
import math

import numpy as np
import jax
import jax.numpy as jnp
from jax import lax
from jax.experimental import pallas as pl
from jax.experimental.pallas import tpu as pltpu
from jax.experimental.pallas import tpu_sc as plsc

_HEADS = 8
_KD = 512
_HALF = 256
_NK = 512
_KNN = 32
_IN = 1024
_VD = 256
_BS = 2048
_TB = 256
_NB = _BS // _TB
_NCAND = 128
_NW = 32
_TPW = _BS // _NW


def _build_cand_tables():
    ci, cj = [], []
    for i in range(_KNN):
        for j in range(min(_KNN, _KNN // (i + 1))):
            ci.append(i)
            cj.append(j)
    n = len(ci)
    assert n <= _NCAND
    p1 = np.zeros((_KNN, _NCAND), np.float32)
    p2 = np.zeros((_KNN, _NCAND), np.float32)
    for c in range(n):
        p1[ci[c], c] = 1.0
        p2[cj[c], c] = 1.0
    bias = np.zeros((_NCAND, 1), np.float32)
    bias[n:, 0] = -1e30
    return p1, p2, bias


_P1, _P2, _BIAS = _build_cand_tables()


def _topk32_t(s, src=None):
    c, t = s.shape
    io = lax.broadcasted_iota(jnp.int32, (c, t), 0)
    lane = lax.broadcasted_iota(jnp.int32, (_KNN, t), 0)
    outv = jnp.zeros((_KNN, t), jnp.float32)
    outi = jnp.zeros((_KNN, t), jnp.float32)
    neg = jnp.float32(float("-inf"))
    for k in range(_KNN):
        m = jnp.max(s, axis=0, keepdims=True)
        am = jnp.min(jnp.where(s == m, io, c), axis=0, keepdims=True)
        hit = io == am
        if src is None:
            pick = am.astype(jnp.float32)
        else:
            pick = jnp.sum(jnp.where(hit, src, 0.0), axis=0, keepdims=True)
        outv = jnp.where(lane == k, m, outv)
        outi = jnp.where(lane == k, pick, outi)
        s = jnp.where(hit, neg, s)
    return outv, outi


def _tc_body(x_ref, wq_ref, bq_ref, keys_ref, p1_ref, p2_ref, bias_ref,
             eye_ref, sc_out_ref, idx_out_ref, sc_scr, idx_scr):
    h = pl.program_id(1)
    hi = lax.Precision.HIGHEST
    q = jnp.dot(x_ref[...], wq_ref[...],
                preferred_element_type=jnp.float32)
    q = q + bq_ref[0]
    k1 = keys_ref[0, 0]
    k2 = keys_ref[0, 1]
    dn = (((1,), (1,)), ((), ()))
    dt = (((1,), (1,)), ((), ()))
    s1t = lax.dot_general(k1, q[:, :_HALF], dt,
                          preferred_element_type=jnp.float32)
    s2t = lax.dot_general(k2, q[:, _HALF:], dt,
                          preferred_element_type=jnp.float32)
    v1t, i1t = _topk32_t(s1t)
    v2t, i2t = _topk32_t(s2t)
    p1 = p1_ref[...]
    p2 = p2_ref[...]
    d0 = (((0,), (0,)), ((), ()))
    cst = lax.dot_general(p1, v1t, d0, precision=hi,
                          preferred_element_type=jnp.float32) \
        + lax.dot_general(p2, v2t, d0, precision=hi,
                          preferred_element_type=jnp.float32) \
        + bias_ref[...]
    cidxt = lax.dot_general(p1, i1t, d0, precision=hi,
                            preferred_element_type=jnp.float32) \
        * jnp.float32(_NK) \
        + lax.dot_general(p2, i2t, d0, precision=hi,
                          preferred_element_type=jnp.float32)
    svt, sit = _topk32_t(cst, src=cidxt)
    e = jnp.exp(svt - svt[0:1])
    smt = e / jnp.sum(e, axis=0, keepdims=True)
    eye = eye_ref[...]
    sm = lax.dot_general(smt, eye, d0, precision=hi,
                         preferred_element_type=jnp.float32)
    si = lax.dot_general(sit, eye, d0, precision=hi,
                         preferred_element_type=jnp.float32)
    sc_scr[h] = sm
    idx_scr[h] = (si + jnp.float32(0.5)).astype(jnp.int32)

    @pl.when(h == _HEADS - 1)
    def _flush():
        sc_all = jnp.concatenate([sc_scr[i] for i in range(_HEADS)], axis=1)
        idx_all = jnp.concatenate([idx_scr[i] for i in range(_HEADS)], axis=1)
        sc_out_ref[...] = sc_all.reshape(2 * _TB, 128)
        idx_out_ref[...] = idx_all.reshape(2 * _TB, 128)


def _tc_topk(x, w_q, bq2, keys, p1, p2, bias):
    f32 = jnp.float32
    return pl.pallas_call(
        _tc_body,
        grid=(_NB, _HEADS),
        in_specs=[
            pl.BlockSpec((_TB, _IN), lambda b, h: (b, 0)),
            pl.BlockSpec((_IN, _KD), lambda b, h: (0, h)),
            pl.BlockSpec((1, 1, _KD), lambda b, h: (h, 0, 0)),
            pl.BlockSpec((1, 2, _NK, _HALF), lambda b, h: (h, 0, 0, 0)),
            pl.BlockSpec((_KNN, _NCAND), lambda b, h: (0, 0)),
            pl.BlockSpec((_KNN, _NCAND), lambda b, h: (0, 0)),
            pl.BlockSpec((_NCAND, 1), lambda b, h: (0, 0)),
            pl.BlockSpec((_KNN, _KNN), lambda b, h: (0, 0)),
        ],
        out_specs=[
            pl.BlockSpec((2 * _TB, 128), lambda b, h: (b, 0)),
            pl.BlockSpec((2 * _TB, 128), lambda b, h: (b, 0)),
        ],
        out_shape=[
            jax.ShapeDtypeStruct((_BS * 2, 128), f32),
            jax.ShapeDtypeStruct((_BS * 2, 128), jnp.int32),
        ],
        scratch_shapes=[
            pltpu.VMEM((_HEADS, _TB, _KNN), f32),
            pltpu.VMEM((_HEADS, _TB, _KNN), jnp.int32),
        ],
    )(x, w_q, bq2, keys, p1, p2, bias, jnp.eye(_KNN, dtype=f32))


def _acc_chunk(buf, acc):
    def body(r, a):
        r4 = r * 4
        for rr in range(4):
            a = tuple(
                a[c] + buf[r4 + rr, pl.ds(c * 16, 16)] for c in range(16)
            )
        return a
    return lax.fori_loop(0, 128 // 4, body, acc)


def _sc_body(values_hbm, idx_hbm, sc_hbm, out_hbm, idx_v, sc_v, buf0, buf1,
             res_v, sem0, sem1):
    nc = 2
    wid = lax.axis_index("s") * nc + lax.axis_index("c")
    base = wid * _TPW
    pltpu.sync_copy(idx_hbm.at[pl.ds(base * 2, _TPW * 2)], idx_v)
    pltpu.sync_copy(sc_hbm.at[pl.ds(base * 2, _TPW * 2)], sc_v)
    pltpu.async_copy(values_hbm.at[idx_v.at[0]], buf0, sem0)

    @pl.loop(0, _TPW)
    def _token(t):
        g = t * 2
        pltpu.async_copy(values_hbm.at[idx_v.at[g + 1]], buf1, sem1)
        pltpu.make_async_copy(values_hbm.at[idx_v.at[g]], buf0, sem0).wait()
        acc0 = tuple(jnp.zeros((16,), jnp.float32) for _ in range(16))
        acc = _acc_chunk(buf0, acc0)

        @pl.when(t < _TPW - 1)
        def _():
            pltpu.async_copy(values_hbm.at[idx_v.at[g + 2]], buf0, sem0)

        pltpu.make_async_copy(values_hbm.at[idx_v.at[g + 1]], buf1, sem1).wait()
        acc = _acc_chunk(buf1, acc)
        for c in range(16):
            res_v[t, pl.ds(c * 16, 16)] = \
                acc[c] * sc_v[g + c // 8, pl.ds((c % 8) * 16, 16)]

    pltpu.sync_copy(res_v, out_hbm.at[pl.ds(base, _TPW)])


def _sc_gather(values, idx, scores):
    mesh = plsc.VectorSubcoreMesh(core_axis_name="c", subcore_axis_name="s")
    f32 = jnp.float32
    call = pl.kernel(
        _sc_body,
        out_type=jax.ShapeDtypeStruct((_BS, _VD), f32),
        mesh=mesh,
        scratch_types=[
            pltpu.VMEM((_TPW * 2, 128), jnp.int32),
            pltpu.VMEM((_TPW * 2, 128), f32),
            pltpu.VMEM((128, _VD), f32),
            pltpu.VMEM((128, _VD), f32),
            pltpu.VMEM((_TPW, _VD), f32),
            pltpu.SemaphoreType.DMA,
            pltpu.SemaphoreType.DMA,
        ],
        compiler_params=pltpu.CompilerParams(use_tc_tiling_on_sc=False),
    )
    return call(values, idx, scores)


def kernel(x, W_q, b_q, keys, values):
    bq2 = b_q.reshape(_HEADS, 1, _KD)
    p1 = jnp.asarray(_P1)
    p2 = jnp.asarray(_P2)
    bias = jnp.asarray(_BIAS)
    sc, idx = _tc_topk(x, W_q, bq2, keys, p1, p2, bias)
    return _sc_gather(values, idx, sc)
